# baseline (speedup 1.0000x reference)
import functools

import jax
import jax.numpy as jnp
from jax import lax
from jax.experimental import pallas as pl
from jax.experimental.pallas import tpu as pltpu
from jax.experimental.pallas import tpu_sc as plsc

N = 10000
E = 320000
D = 128
ALPHA = 0.5

NC = 2
NS = 16
NW = NC * NS
B = 128
NBUF = 2
T = 40
PH = 4
IDX_ROWS = NBUF * T + 2 * PH * NBUF
CHUNK = NBUF * T * B
E_PAD = NW * IDX_ROWS * B
ACC_ROWS = 10240
RPS = ACC_ROWS // NS
TRASH = N


def _sc_agg(table, src2d, dst2d):
    mesh = plsc.VectorSubcoreMesh(core_axis_name="c", subcore_axis_name="s")

    @functools.partial(
        pl.kernel,
        out_type=jax.ShapeDtypeStruct((NC, ACC_ROWS, D), jnp.float32),
        mesh=mesh,
        scratch_types=[
            pltpu.VMEM((2, PH * NBUF, B), jnp.int32),
            pltpu.VMEM((2, PH * NBUF, B), jnp.int32),
            pltpu.VMEM((NBUF * B, D), jnp.float32),
            pltpu.VMEM((16, D), jnp.float32),
            pltpu.VMEM_SHARED((ACC_ROWS, D), jnp.float32),
            [pltpu.SemaphoreType.DMA] * NBUF,
            [pltpu.SemaphoreType.DMA] * NBUF,
        ],
    )
    def k(table_h, src_h, dst_h, out_h, srcv, dstv, rows, zb, acc, gs, ss):
        c = lax.axis_index("c")
        s = lax.axis_index("s")
        w = c * NS + s
        wbase = w * IDX_ROWS

        def gather_desc(p, r, b):
            return pltpu.make_async_copy(table_h.at[srcv.at[p, r]],
                                         rows.at[pl.ds(b * B, B)], gs[b])

        zv = jnp.zeros((16,), jnp.float32)
        for i in range(16):
            for j in range(D // 16):
                zb[i, pl.ds(j * 16, 16)] = zv
        nz = RPS // 16
        zd = [None] * NBUF
        for t in range(nz):
            if zd[t % NBUF] is not None:
                zd[t % NBUF].wait()
            zd[t % NBUF] = pltpu.async_copy(
                zb, acc.at[pl.ds(s * RPS + t * 16, 16)], gs[t % NBUF])
        for d in zd:
            d.wait()
        plsc.subcore_barrier()

        rpp = PH * NBUF
        pltpu.sync_copy(src_h.at[pl.ds(wbase, rpp)], srcv.at[0])
        pltpu.sync_copy(dst_h.at[pl.ds(wbase, rpp)], dstv.at[0])
        pltpu.sync_copy(src_h.at[pl.ds(wbase + rpp, rpp)], srcv.at[1])
        pltpu.sync_copy(dst_h.at[pl.ds(wbase + rpp, rpp)], dstv.at[1])
        for b in range(NBUF):
            pltpu.async_copy(table_h.at[srcv.at[0, b]],
                             rows.at[pl.ds(b * B, B)], gs[b])

        def body(t, carry):
            p = lax.rem(t // PH, 2)
            tm = lax.rem(t, PH)
            p1 = lax.rem((t + 1) // PH, 2)
            tm1 = lax.rem(t + 1, PH)
            sdesc = []
            for b in range(NBUF):
                gather_desc(p, tm * NBUF + b, b).wait()
                sdesc.append(pltpu.async_copy(
                    rows.at[pl.ds(b * B, B)],
                    acc.at[dstv.at[p, tm * NBUF + b]], ss[b], add=True))
            for b in range(NBUF):
                sdesc[b].wait()
                pltpu.async_copy(table_h.at[srcv.at[p1, tm1 * NBUF + b]],
                                 rows.at[pl.ds(b * B, B)], gs[b])

            @pl.when(tm == PH - 1)
            def _prefetch():
                off = wbase + (t // PH + 2) * rpp
                pltpu.sync_copy(src_h.at[pl.ds(off, rpp)], srcv.at[p])
                pltpu.sync_copy(dst_h.at[pl.ds(off, rpp)], dstv.at[p])
            return carry

        lax.fori_loop(0, T, body, 0)
        pT = (T // PH) % 2
        for b in range(NBUF):
            gather_desc(pT, (T % PH) * NBUF + b, b).wait()
        plsc.subcore_barrier()
        pltpu.sync_copy(acc.at[pl.ds(s * RPS, RPS)],
                        out_h.at[c, pl.ds(s * RPS, RPS)])

    return k(table, src2d, dst2d)


def _sc_degree(dst2d):
    mesh = plsc.VectorSubcoreMesh(core_axis_name="c", subcore_axis_name="s")

    @functools.partial(
        pl.kernel,
        out_type=jax.ShapeDtypeStruct((NC, ACC_ROWS, 16), jnp.float32),
        mesh=mesh,
        scratch_types=[
            pltpu.VMEM((NBUF * T, B), jnp.int32),
            pltpu.VMEM((B, 16), jnp.float32),
            pltpu.VMEM((16, 16), jnp.float32),
            pltpu.VMEM_SHARED((ACC_ROWS, 16), jnp.float32),
            [pltpu.SemaphoreType.DMA] * NBUF,
        ],
    )
    def k(dst_h, out_h, dstv, ones, zb, acc, ss):
        c = lax.axis_index("c")
        s = lax.axis_index("s")
        w = c * NS + s
        ov = jnp.ones((16,), jnp.float32)
        zv = jnp.zeros((16,), jnp.float32)
        for i in range(B):
            ones[i, pl.ds(0, 16)] = ov
        for i in range(16):
            zb[i, pl.ds(0, 16)] = zv

        def scatter_desc(r, b):
            return pltpu.make_async_copy(ones, acc.at[dstv.at[r]], ss[b])

        zd = [None] * NBUF
        for t in range(RPS // 16):
            if zd[t % NBUF] is not None:
                zd[t % NBUF].wait()
            zd[t % NBUF] = pltpu.async_copy(
                zb, acc.at[pl.ds(s * RPS + t * 16, 16)], ss[t % NBUF])
        for d in zd:
            d.wait()
        pltpu.sync_copy(dst_h.at[pl.ds(w * IDX_ROWS, NBUF * T)], dstv)
        plsc.subcore_barrier()

        for b in range(NBUF):
            pltpu.async_copy(ones, acc.at[dstv.at[b]], ss[b], add=True)

        def body(t, carry):
            for b in range(NBUF):
                scatter_desc((t - 1) * NBUF + b, b).wait()
                pltpu.async_copy(ones, acc.at[dstv.at[t * NBUF + b]],
                                 ss[b], add=True)
            return carry

        lax.fori_loop(1, T, body, 0)
        for b in range(NBUF):
            scatter_desc((T - 1) * NBUF + b, b).wait()
        plsc.subcore_barrier()
        pltpu.sync_copy(acc.at[pl.ds(s * RPS, RPS)],
                        out_h.at[c, pl.ds(s * RPS, RPS)])

    return k(dst2d)


_R = 1000


def _tc_dinv(dparts):
    def body(dp_ref, o_ref):
        dp = dp_ref[...]
        deg = dp[0, :, 0:1] + dp[1, :, 0:1] + 1.0
        o_ref[...] = jnp.broadcast_to(lax.rsqrt(deg), (_R, D))

    return pl.pallas_call(
        body,
        grid=(N // _R,),
        in_specs=[pl.BlockSpec((NC, _R, 16), lambda i: (0, i, 0))],
        out_specs=pl.BlockSpec((_R, D), lambda i: (i, 0)),
        out_shape=jax.ShapeDtypeStruct((N, D), jnp.float32),
    )(dparts)


def _tc_first(x, W, dinvb):
    def body(x_ref, w_ref, dv_ref, o_ref):
        h = lax.dot_general(x_ref[...], w_ref[...], (((1,), (1,)), ((), ())),
                            preferred_element_type=jnp.float32)
        o_ref[...] = dv_ref[...] * h

    return pl.pallas_call(
        body,
        grid=(N // _R,),
        in_specs=[
            pl.BlockSpec((_R, D), lambda i: (i, 0)),
            pl.BlockSpec((D, D), lambda i: (0, 0)),
            pl.BlockSpec((_R, D), lambda i: (i, 0)),
        ],
        out_specs=pl.BlockSpec((_R, D), lambda i: (i, 0)),
        out_shape=jax.ShapeDtypeStruct((N, D), jnp.float32),
    )(x, W, dinvb)


def _tc_mid(p, hp, xres, dinvb, b, g, be, Wn):
    def body(p_ref, hp_ref, xr_ref, dv_ref, b_ref, g_ref, be_ref, wn_ref,
             xn_ref, hn_ref):
        pv = p_ref[...]
        dinv = dv_ref[...]
        conv = dinv * (pv[0] + pv[1] + hp_ref[...]) + b_ref[...]
        mu = jnp.mean(conv, axis=-1, keepdims=True)
        var = jnp.mean((conv - mu) ** 2, axis=-1, keepdims=True)
        ln = (conv - mu) / jnp.sqrt(var + 1e-5) * g_ref[...] + be_ref[...]
        xn = jnp.maximum(ALPHA * ln + (1.0 - ALPHA) * xr_ref[...], 0.0)
        xn_ref[...] = xn
        hw = lax.dot_general(xn, wn_ref[...], (((1,), (1,)), ((), ())),
                             preferred_element_type=jnp.float32)
        hn_ref[...] = dinv * hw

    return pl.pallas_call(
        body,
        grid=(N // _R,),
        in_specs=[
            pl.BlockSpec((NC, _R, D), lambda i: (0, i, 0)),
            pl.BlockSpec((_R, D), lambda i: (i, 0)),
            pl.BlockSpec((_R, D), lambda i: (i, 0)),
            pl.BlockSpec((_R, D), lambda i: (i, 0)),
            pl.BlockSpec((1, D), lambda i: (0, 0)),
            pl.BlockSpec((1, D), lambda i: (0, 0)),
            pl.BlockSpec((1, D), lambda i: (0, 0)),
            pl.BlockSpec((D, D), lambda i: (0, 0)),
        ],
        out_specs=(
            pl.BlockSpec((_R, D), lambda i: (i, 0)),
            pl.BlockSpec((_R, D), lambda i: (i, 0)),
        ),
        out_shape=(
            jax.ShapeDtypeStruct((N, D), jnp.float32),
            jax.ShapeDtypeStruct((N, D), jnp.float32),
        ),
    )(p, hp, xres, dinvb, b, g, be, Wn)


def _tc_last(p, hp, dinvb, b):
    def body(p_ref, hp_ref, dv_ref, b_ref, o_ref):
        pv = p_ref[...]
        o_ref[...] = dv_ref[...] * (pv[0] + pv[1] + hp_ref[...]) + b_ref[...]

    return pl.pallas_call(
        body,
        grid=(N // _R,),
        in_specs=[
            pl.BlockSpec((NC, _R, D), lambda i: (0, i, 0)),
            pl.BlockSpec((_R, D), lambda i: (i, 0)),
            pl.BlockSpec((_R, D), lambda i: (i, 0)),
            pl.BlockSpec((1, D), lambda i: (0, 0)),
        ],
        out_specs=pl.BlockSpec((_R, D), lambda i: (i, 0)),
        out_shape=jax.ShapeDtypeStruct((N, D), jnp.float32),
    )(p, hp, dinvb, b)


def kernel(x, edge_index, W1, b1, g1, be1, W2, b2, g2, be2, W3, b3):
    pad = NW * CHUNK - E
    srcp = jnp.concatenate([edge_index[0], jnp.zeros((pad,), jnp.int32)])
    dstp = jnp.concatenate([edge_index[1], jnp.full((pad,), TRASH, jnp.int32)])
    slack = IDX_ROWS - NBUF * T
    src2d = jnp.concatenate(
        [srcp.reshape(NW, NBUF * T, B),
         jnp.zeros((NW, slack, B), jnp.int32)], axis=1
    ).reshape(NW * IDX_ROWS, B)
    dst2d = jnp.concatenate(
        [dstp.reshape(NW, NBUF * T, B),
         jnp.full((NW, slack, B), TRASH, jnp.int32)], axis=1
    ).reshape(NW * IDX_ROWS, B)

    dparts = _sc_degree(dst2d)
    dinvb = _tc_dinv(dparts)

    h1p = _tc_first(x, W1, dinvb)
    p1 = _sc_agg(h1p, src2d, dst2d)
    x1, h2p = _tc_mid(p1, h1p, x, dinvb, b1.reshape(1, D), g1.reshape(1, D),
                      be1.reshape(1, D), W2)
    p2 = _sc_agg(h2p, src2d, dst2d)
    _, h3p = _tc_mid(p2, h2p, x1, dinvb, b2.reshape(1, D), g2.reshape(1, D),
                     be2.reshape(1, D), W3)
    p3 = _sc_agg(h3p, src2d, dst2d)
    return _tc_last(p3, h3p, dinvb, b3.reshape(1, D))


# speedup vs baseline: 1.5856x; 1.5856x over previous
import functools

import jax
import jax.numpy as jnp
from jax import lax
from jax.experimental import pallas as pl
from jax.experimental.pallas import tpu as pltpu
from jax.experimental.pallas import tpu_sc as plsc

N = 10000
E = 320000
D = 128
ALPHA = 0.5

NC = 2
NS = 16
NW = NC * NS
B = 128
NBUF = 2
PB = 8
NPH = 10
BATCHES = PB * NPH
IDX_ROWS = BATCHES
CHUNK = BATCHES * B
E_PAD = NW * IDX_ROWS * B
ACC_ROWS = 10240
RPS = ACC_ROWS // NS
TRASH = N


def _sc_agg(table, src2d, dst2d):
    mesh = plsc.VectorSubcoreMesh(core_axis_name="c", subcore_axis_name="s")

    @functools.partial(
        pl.kernel,
        out_type=jax.ShapeDtypeStruct((NC, ACC_ROWS, D), jnp.float32),
        mesh=mesh,
        scratch_types=[
            pltpu.VMEM((PB, B), jnp.int32),
            pltpu.VMEM((PB, B), jnp.int32),
            pltpu.VMEM((NBUF * B, D), jnp.float32),
            pltpu.VMEM((16, D), jnp.float32),
            pltpu.VMEM_SHARED((ACC_ROWS, D), jnp.float32),
            [pltpu.SemaphoreType.DMA] * NBUF,
            [pltpu.SemaphoreType.DMA] * NBUF,
        ],
    )
    def k(table_h, src_h, dst_h, out_h, srcv, dstv, rows, zb, acc, gs, ss):
        c = lax.axis_index("c")
        s = lax.axis_index("s")
        w = c * NS + s
        wbase = w * IDX_ROWS

        zv = jnp.zeros((16,), jnp.float32)
        for i in range(16):
            for j in range(D // 16):
                zb[i, pl.ds(j * 16, 16)] = zv
        nz = RPS // 16
        zd = [None] * NBUF
        for t in range(nz):
            if zd[t % NBUF] is not None:
                zd[t % NBUF].wait()
            zd[t % NBUF] = pltpu.async_copy(
                zb, acc.at[pl.ds(s * RPS + t * 16, 16)], gs[t % NBUF])
        for d in zd:
            d.wait()
        plsc.subcore_barrier()

        def body(ph, carry):
            pltpu.sync_copy(src_h.at[pl.ds(wbase + ph * PB, PB)], srcv)
            pltpu.sync_copy(dst_h.at[pl.ds(wbase + ph * PB, PB)], dstv)
            pltpu.async_copy(table_h.at[srcv.at[0]],
                             rows.at[pl.ds(0, B)], gs[0])
            for r in range(PB):
                b = r % NBUF
                pltpu.make_async_copy(table_h.at[srcv.at[r]],
                                      rows.at[pl.ds(b * B, B)], gs[b]).wait()
                if r + 1 < PB:
                    b1 = (r + 1) % NBUF
                    pltpu.async_copy(table_h.at[srcv.at[r + 1]],
                                     rows.at[pl.ds(b1 * B, B)], gs[b1])
                pltpu.sync_copy(rows.at[pl.ds(b * B, B)],
                                acc.at[dstv.at[r]], add=True)
            return carry

        lax.fori_loop(0, NPH, body, 0)
        plsc.subcore_barrier()
        pltpu.sync_copy(acc.at[pl.ds(s * RPS, RPS)],
                        out_h.at[c, pl.ds(s * RPS, RPS)])

    return k(table, src2d, dst2d)


def _sc_degree(dst2d):
    mesh = plsc.VectorSubcoreMesh(core_axis_name="c", subcore_axis_name="s")

    @functools.partial(
        pl.kernel,
        out_type=jax.ShapeDtypeStruct((NC, ACC_ROWS, 16), jnp.float32),
        mesh=mesh,
        scratch_types=[
            pltpu.VMEM((BATCHES, B), jnp.int32),
            pltpu.VMEM((B, 16), jnp.float32),
            pltpu.VMEM((16, 16), jnp.float32),
            pltpu.VMEM_SHARED((ACC_ROWS, 16), jnp.float32),
            [pltpu.SemaphoreType.DMA] * NBUF,
        ],
    )
    def k(dst_h, out_h, dstv, ones, zb, acc, ss):
        c = lax.axis_index("c")
        s = lax.axis_index("s")
        w = c * NS + s
        ov = jnp.ones((16,), jnp.float32)
        zv = jnp.zeros((16,), jnp.float32)
        for i in range(B):
            ones[i, pl.ds(0, 16)] = ov
        for i in range(16):
            zb[i, pl.ds(0, 16)] = zv

        def scatter_desc(r, b):
            return pltpu.make_async_copy(ones, acc.at[dstv.at[r]], ss[b])

        zd = [None] * NBUF
        for t in range(RPS // 16):
            if zd[t % NBUF] is not None:
                zd[t % NBUF].wait()
            zd[t % NBUF] = pltpu.async_copy(
                zb, acc.at[pl.ds(s * RPS + t * 16, 16)], ss[t % NBUF])
        for d in zd:
            d.wait()
        pltpu.sync_copy(dst_h.at[pl.ds(w * IDX_ROWS, BATCHES)], dstv)
        plsc.subcore_barrier()

        for b in range(NBUF):
            pltpu.async_copy(ones, acc.at[dstv.at[b]], ss[b], add=True)

        def body(t, carry):
            for b in range(NBUF):
                scatter_desc((t - 1) * NBUF + b, b).wait()
                pltpu.async_copy(ones, acc.at[dstv.at[t * NBUF + b]],
                                 ss[b], add=True)
            return carry

        lax.fori_loop(1, BATCHES // NBUF, body, 0)
        for b in range(NBUF):
            scatter_desc(BATCHES - NBUF + b, b).wait()
        plsc.subcore_barrier()
        pltpu.sync_copy(acc.at[pl.ds(s * RPS, RPS)],
                        out_h.at[c, pl.ds(s * RPS, RPS)])

    return k(dst2d)


_R = 1000


def _tc_dinv(dparts):
    def body(dp_ref, o_ref):
        dp = dp_ref[...]
        deg = dp[0, :, 0:1] + dp[1, :, 0:1] + 1.0
        o_ref[...] = jnp.broadcast_to(lax.rsqrt(deg), (_R, D))

    return pl.pallas_call(
        body,
        grid=(N // _R,),
        in_specs=[pl.BlockSpec((NC, _R, 16), lambda i: (0, i, 0))],
        out_specs=pl.BlockSpec((_R, D), lambda i: (i, 0)),
        out_shape=jax.ShapeDtypeStruct((N, D), jnp.float32),
    )(dparts)


def _tc_first(x, W, dinvb):
    def body(x_ref, w_ref, dv_ref, o_ref):
        h = lax.dot_general(x_ref[...], w_ref[...], (((1,), (1,)), ((), ())),
                            preferred_element_type=jnp.float32)
        o_ref[...] = dv_ref[...] * h

    return pl.pallas_call(
        body,
        grid=(N // _R,),
        in_specs=[
            pl.BlockSpec((_R, D), lambda i: (i, 0)),
            pl.BlockSpec((D, D), lambda i: (0, 0)),
            pl.BlockSpec((_R, D), lambda i: (i, 0)),
        ],
        out_specs=pl.BlockSpec((_R, D), lambda i: (i, 0)),
        out_shape=jax.ShapeDtypeStruct((N, D), jnp.float32),
    )(x, W, dinvb)


def _tc_mid(p, hp, xres, dinvb, b, g, be, Wn):
    def body(p_ref, hp_ref, xr_ref, dv_ref, b_ref, g_ref, be_ref, wn_ref,
             xn_ref, hn_ref):
        pv = p_ref[...]
        dinv = dv_ref[...]
        conv = dinv * (pv[0] + pv[1] + hp_ref[...]) + b_ref[...]
        mu = jnp.mean(conv, axis=-1, keepdims=True)
        var = jnp.mean((conv - mu) ** 2, axis=-1, keepdims=True)
        ln = (conv - mu) / jnp.sqrt(var + 1e-5) * g_ref[...] + be_ref[...]
        xn = jnp.maximum(ALPHA * ln + (1.0 - ALPHA) * xr_ref[...], 0.0)
        xn_ref[...] = xn
        hw = lax.dot_general(xn, wn_ref[...], (((1,), (1,)), ((), ())),
                             preferred_element_type=jnp.float32)
        hn_ref[...] = dinv * hw

    return pl.pallas_call(
        body,
        grid=(N // _R,),
        in_specs=[
            pl.BlockSpec((NC, _R, D), lambda i: (0, i, 0)),
            pl.BlockSpec((_R, D), lambda i: (i, 0)),
            pl.BlockSpec((_R, D), lambda i: (i, 0)),
            pl.BlockSpec((_R, D), lambda i: (i, 0)),
            pl.BlockSpec((1, D), lambda i: (0, 0)),
            pl.BlockSpec((1, D), lambda i: (0, 0)),
            pl.BlockSpec((1, D), lambda i: (0, 0)),
            pl.BlockSpec((D, D), lambda i: (0, 0)),
        ],
        out_specs=(
            pl.BlockSpec((_R, D), lambda i: (i, 0)),
            pl.BlockSpec((_R, D), lambda i: (i, 0)),
        ),
        out_shape=(
            jax.ShapeDtypeStruct((N, D), jnp.float32),
            jax.ShapeDtypeStruct((N, D), jnp.float32),
        ),
    )(p, hp, xres, dinvb, b, g, be, Wn)


def _tc_last(p, hp, dinvb, b):
    def body(p_ref, hp_ref, dv_ref, b_ref, o_ref):
        pv = p_ref[...]
        o_ref[...] = dv_ref[...] * (pv[0] + pv[1] + hp_ref[...]) + b_ref[...]

    return pl.pallas_call(
        body,
        grid=(N // _R,),
        in_specs=[
            pl.BlockSpec((NC, _R, D), lambda i: (0, i, 0)),
            pl.BlockSpec((_R, D), lambda i: (i, 0)),
            pl.BlockSpec((_R, D), lambda i: (i, 0)),
            pl.BlockSpec((1, D), lambda i: (0, 0)),
        ],
        out_specs=pl.BlockSpec((_R, D), lambda i: (i, 0)),
        out_shape=jax.ShapeDtypeStruct((N, D), jnp.float32),
    )(p, hp, dinvb, b)


def kernel(x, edge_index, W1, b1, g1, be1, W2, b2, g2, be2, W3, b3):
    pad = E_PAD - E
    srcp = jnp.concatenate([edge_index[0], jnp.zeros((pad,), jnp.int32)])
    dstp = jnp.concatenate([edge_index[1], jnp.full((pad,), TRASH, jnp.int32)])
    src2d = srcp.reshape(NW * IDX_ROWS, B)
    dst2d = dstp.reshape(NW * IDX_ROWS, B)

    dparts = _sc_degree(dst2d)
    dinvb = _tc_dinv(dparts)

    h1p = _tc_first(x, W1, dinvb)
    p1 = _sc_agg(h1p, src2d, dst2d)
    x1, h2p = _tc_mid(p1, h1p, x, dinvb, b1.reshape(1, D), g1.reshape(1, D),
                      be1.reshape(1, D), W2)
    p2 = _sc_agg(h2p, src2d, dst2d)
    _, h3p = _tc_mid(p2, h2p, x1, dinvb, b2.reshape(1, D), g2.reshape(1, D),
                     be2.reshape(1, D), W3)
    p3 = _sc_agg(h3p, src2d, dst2d)
    return _tc_last(p3, h3p, dinvb, b3.reshape(1, D))
